# baseline (speedup 1.0000x reference)
import functools

import jax
import jax.numpy as jnp
from jax import lax
from jax.experimental import pallas as pl
from jax.experimental.pallas import tpu as pltpu
from jax.experimental.pallas import tpu_sc as plsc

VOCAB = 100000
D = 128
B = 16384

NUM_CORES = 2
NUM_SUBCORES = 16
LANES = 16
NW = NUM_CORES * NUM_SUBCORES
BPW = B // NW
CHUNK = 128
NCHUNK = BPW // CHUNK

_mesh = plsc.VectorSubcoreMesh(core_axis_name="c", subcore_axis_name="s")


@functools.partial(
    pl.kernel,
    mesh=_mesh,
    out_type=jax.ShapeDtypeStruct((B,), jnp.float32),
    compiler_params=pltpu.CompilerParams(needs_layout_passes=False),
    scratch_types=[
        pltpu.VMEM((BPW,), jnp.int32),
        pltpu.VMEM((BPW,), jnp.int32),
        pltpu.VMEM((2, CHUNK, D), jnp.float32),
        pltpu.VMEM((2, CHUNK, D), jnp.float32),
        pltpu.VMEM((BPW,), jnp.float32),
        pltpu.SemaphoreType.DMA,
        pltpu.SemaphoreType.DMA,
    ],
)
def _sc_skipgram(cw_hbm, xw_hbm, tin_hbm, tout_hbm, out_hbm,
                 ci_v, xi_v, a_v, b_v, o_v, sem_a, sem_b):
    wid = lax.axis_index("s") * NUM_CORES + lax.axis_index("c")
    base = wid * BPW
    lane = lax.iota(jnp.int32, LANES)

    pltpu.sync_copy(cw_hbm.at[pl.ds(base, BPW)], ci_v)
    pltpu.sync_copy(xw_hbm.at[pl.ds(base, BPW)], xi_v)

    def fire(c, slot):
        cp_a = pltpu.async_copy(
            tin_hbm.at[ci_v.at[pl.ds(c * CHUNK, CHUNK)]], a_v.at[slot], sem_a)
        cp_b = pltpu.async_copy(
            tout_hbm.at[xi_v.at[pl.ds(c * CHUNK, CHUNK)]], b_v.at[slot], sem_b)
        return cp_a, cp_b

    pend = fire(0, 0)
    for c in range(NCHUNK):
        cur = c % 2
        pend[0].wait()
        pend[1].wait()
        if c + 1 < NCHUNK:
            pend = fire(c + 1, 1 - cur)

        for g in range(CHUNK // LANES):
            rows = g * LANES + lane

            def body(j, accs):
                acc0, acc1 = accs
                col0 = jnp.full((LANES,), 2 * j, jnp.int32)
                col1 = col0 + 1
                va0 = plsc.load_gather(a_v.at[cur], [rows, col0])
                vb0 = plsc.load_gather(b_v.at[cur], [rows, col0])
                va1 = plsc.load_gather(a_v.at[cur], [rows, col1])
                vb1 = plsc.load_gather(b_v.at[cur], [rows, col1])
                return acc0 + va0 * vb0, acc1 + va1 * vb1

            z = jnp.zeros((LANES,), jnp.float32)
            acc0, acc1 = lax.fori_loop(0, D // 2, body, (z, z), unroll=8)
            o_v[pl.ds(c * CHUNK + g * LANES, LANES)] = acc0 + acc1

    pltpu.sync_copy(o_v, out_hbm.at[pl.ds(base, BPW)])


def kernel(center_words, context_words, input_embeddings, output_embeddings):
    return _sc_skipgram(center_words.astype(jnp.int32),
                        context_words.astype(jnp.int32),
                        input_embeddings, output_embeddings)


# speedup vs baseline: 2.5714x; 2.5714x over previous
import functools

import jax
import jax.numpy as jnp
from jax import lax
from jax.experimental import pallas as pl
from jax.experimental.pallas import tpu as pltpu
from jax.experimental.pallas import tpu_sc as plsc

VOCAB = 100000
D = 128
B = 16384

NUM_CORES = 2
NUM_SUBCORES = 16
LANES = 16
NW = NUM_CORES * NUM_SUBCORES
BPW = B // NW
CHUNK = 128
NCHUNK = BPW // CHUNK

_mesh = plsc.VectorSubcoreMesh(core_axis_name="c", subcore_axis_name="s")


@functools.partial(
    pl.kernel,
    mesh=_mesh,
    out_type=jax.ShapeDtypeStruct((B,), jnp.float32),
    compiler_params=pltpu.CompilerParams(needs_layout_passes=False),
    scratch_types=[
        pltpu.VMEM((BPW,), jnp.int32),
        pltpu.VMEM((BPW,), jnp.int32),
        pltpu.VMEM((2, CHUNK, D), jnp.float32),
        pltpu.VMEM((2, CHUNK, D), jnp.float32),
        pltpu.VMEM((BPW,), jnp.float32),
        pltpu.VMEM((LANES * 17,), jnp.float32),
        pltpu.SemaphoreType.DMA,
        pltpu.SemaphoreType.DMA,
    ],
)
def _sc_skipgram(cw_hbm, xw_hbm, tin_hbm, tout_hbm, out_hbm,
                 ci_v, xi_v, a_v, b_v, o_v, t_v, sem_a, sem_b):
    wid = lax.axis_index("s") * NUM_CORES + lax.axis_index("c")
    base = wid * BPW
    lane = lax.iota(jnp.int32, LANES)
    lane17 = lane * 17

    pltpu.sync_copy(cw_hbm.at[pl.ds(base, BPW)], ci_v)
    pltpu.sync_copy(xw_hbm.at[pl.ds(base, BPW)], xi_v)

    def fire(c, slot):
        cp_a = pltpu.async_copy(
            tin_hbm.at[ci_v.at[pl.ds(c * CHUNK, CHUNK)]], a_v.at[slot], sem_a)
        cp_b = pltpu.async_copy(
            tout_hbm.at[xi_v.at[pl.ds(c * CHUNK, CHUNK)]], b_v.at[slot], sem_b)
        return cp_a, cp_b

    pend = fire(0, 0)
    for c in range(NCHUNK):
        cur = c % 2
        pend[0].wait()
        pend[1].wait()
        if c + 1 < NCHUNK:
            pend = fire(c + 1, 1 - cur)

        def group_body(g, _):
            for rl in range(LANES):
                r = g * LANES + rl
                p = [a_v[cur, r, pl.ds(j * LANES, LANES)] *
                     b_v[cur, r, pl.ds(j * LANES, LANES)]
                     for j in range(D // LANES)]
                s = ((p[0] + p[1]) + (p[2] + p[3])) + \
                    ((p[4] + p[5]) + (p[6] + p[7]))
                plsc.store_scatter(t_v, [lane17 + rl], s)
            q = [t_v[pl.ds(cc * 17, LANES)] for cc in range(LANES)]
            while len(q) > 1:
                q = [q[2 * i] + q[2 * i + 1] for i in range(len(q) // 2)]
            o_v[pl.ds(c * CHUNK + g * LANES, LANES)] = q[0]
            return 0

        lax.fori_loop(0, CHUNK // LANES, group_body, 0)

    pltpu.sync_copy(o_v, out_hbm.at[pl.ds(base, BPW)])


def kernel(center_words, context_words, input_embeddings, output_embeddings):
    return _sc_skipgram(center_words.astype(jnp.int32),
                        context_words.astype(jnp.int32),
                        input_embeddings, output_embeddings)
